# Initial kernel scaffold; baseline (speedup 1.0000x reference)
import jax
import jax.numpy as jnp
from jax.experimental import pallas as pl


def kernel(x, alpha, norm_w, norm_b, W_in, b_in, qkv_W, qkv_b, conv_w, A, Bp, Cp, W_out, b_out):
    raise NotImplementedError("write your pallas kernel here")


# speedup vs baseline: 5.9394x; 5.9394x over previous
import functools

import jax
import jax.numpy as jnp
from jax import lax
from jax.experimental import pallas as pl
from jax.experimental.pallas import tpu as pltpu
from jax.experimental.pallas import tpu_sc as plsc

B_SZ = 2
L = 4096
BL = B_SZ * L
DIM = 1024
ED = 336
NH = 6
HD = 56
HDP = 64
EDP = NH * HDP
DS = 16
K_CH = 819
K_PAD = 896
ROWS_W = 56
NW = 32

_F32 = jnp.float32


def _front_body(x_ref, winT_ref, bin_ref, wqT_ref, wkT_ref, wvT_ref,
                bq_ref, bk_ref, bv_ref, alpha_ref, nw_ref, nb_ref,
                out_ref, ch2_ref):
    x = x_ref[...]
    xn = jnp.tanh(x * alpha_ref[...]) * nw_ref[...] + nb_ref[...]
    xp = jnp.dot(xn, winT_ref[...], preferred_element_type=_F32) + bin_ref[...]
    n2 = jnp.sum(xp * xp, axis=1, keepdims=True)
    xp = xp / jnp.maximum(jnp.sqrt(n2), 1e-12)
    q = jnp.dot(xp, wqT_ref[...], preferred_element_type=_F32) + bq_ref[...]
    k = jnp.dot(xp, wkT_ref[...], preferred_element_type=_F32) + bk_ref[...]
    v = jnp.dot(xp, wvT_ref[...], preferred_element_type=_F32) + bv_ref[...]
    outs = []
    for h in range(NH):
        qh = q[:, HDP * h:HDP * (h + 1)]
        best_a = jnp.sum(qh * k[:, 0:HDP], axis=1, keepdims=True)
        out_h = v[:, 0:HDP]
        for j in range(1, NH):
            kj = k[:, HDP * j:HDP * (j + 1)]
            a = jnp.sum(qh * kj, axis=1, keepdims=True)
            upd = a > best_a
            best_a = jnp.where(upd, a, best_a)
            out_h = jnp.where(upd, v[:, HDP * j:HDP * (j + 1)], out_h)
        outs.append(out_h)
    out = jnp.concatenate(outs, axis=1)
    out_ref[...] = out
    ch2_ref[...] = jnp.sum(out * out, axis=1)[None, None, :]


def _run_front(x2d, winT, bin_p, wqT, wkT, wvT, bq, bk, bv, alpha11, nw, nb):
    ntiles = 16
    rows = BL // ntiles
    full = lambda shape: pl.BlockSpec(shape, lambda i: (0,) * len(shape))
    out, ch2 = pl.pallas_call(
        _front_body,
        grid=(ntiles,),
        in_specs=[
            pl.BlockSpec((rows, DIM), lambda i: (i, 0)),
            full((DIM, EDP)), full((1, EDP)),
            full((EDP, EDP)), full((EDP, EDP)), full((EDP, EDP)),
            full((1, EDP)), full((1, EDP)), full((1, EDP)),
            full((1, 1)), full((1, DIM)), full((1, DIM)),
        ],
        out_specs=[
            pl.BlockSpec((rows, EDP), lambda i: (i, 0)),
            pl.BlockSpec((1, 1, rows), lambda i: (i, 0, 0)),
        ],
        out_shape=[
            jax.ShapeDtypeStruct((BL, EDP), _F32),
            jax.ShapeDtypeStruct((ntiles, 1, rows), _F32),
        ],
    )(x2d, winT, bin_p, wqT, wkT, wvT, bq, bk, bv, alpha11, nw, nb)
    return out, ch2.reshape(B_SZ, L)


def _sc_gather(table, idx):
    mesh = plsc.VectorSubcoreMesh(core_axis_name="c", subcore_axis_name="s")

    @functools.partial(
        pl.kernel,
        out_type=jax.ShapeDtypeStruct((K_PAD, B_SZ * EDP), _F32),
        mesh=mesh,
        scratch_types=[
            pltpu.VMEM((ROWS_W,), jnp.int32),
            pltpu.VMEM((ROWS_W, EDP), _F32),
            pltpu.SemaphoreType.DMA,
        ],
    )
    def gather_k(table_hbm, idx_hbm, out_hbm, idx_v, rows_v, sem):
        wid = lax.axis_index("s") * 2 + lax.axis_index("c")
        b = wid // 16
        t0 = (wid % 16) * ROWS_W
        pltpu.sync_copy(idx_hbm.at[pl.ds(wid * ROWS_W, ROWS_W)], idx_v)
        pltpu.async_copy(table_hbm.at[idx_v], rows_v, sem).wait()
        pltpu.sync_copy(rows_v,
                        out_hbm.at[pl.ds(t0, ROWS_W), pl.ds(b * EDP, EDP)])

    return gather_k(table, idx)


def _scan_body(u_ref, cw_ref, sigb_ref, sigc_ref, a_ref, woT_ref, bout_ref,
               xproc_ref, xc_scr, y_scr):
    u = u_ref[...]
    w = cw_ref[...]
    z1 = jnp.zeros((1, EDP), _F32)
    z2 = jnp.zeros((2, EDP), _F32)
    z3 = jnp.zeros((3, EDP), _F32)
    xc = u * w[3:4, :]
    xc = xc + jnp.concatenate([z1, u[:-1, :]], axis=0) * w[2:3, :]
    xc = xc + jnp.concatenate([z2, u[:-2, :]], axis=0) * w[1:2, :]
    xc = xc + jnp.concatenate([z3, u[:-3, :]], axis=0) * w[0:1, :]
    xc_scr[...] = xc
    A = a_ref[...]
    sigb = sigb_ref[...]
    sigc = sigc_ref[...]

    def step(t, h):
        ut = xc_scr[pl.ds(t, 1), :]
        h = jnp.dot(A, h, preferred_element_type=_F32) + sigb * ut
        y_scr[pl.ds(t, 1), :] = jnp.sum(h * sigc, axis=0, keepdims=True)
        return h

    lax.fori_loop(0, K_CH, step, jnp.zeros((DS, EDP), _F32))
    y = y_scr[...]
    xp = jnp.dot(y, woT_ref[...], preferred_element_type=_F32) + bout_ref[...]
    mask = lax.broadcasted_iota(jnp.int32, (K_PAD, 1), 0) < K_CH
    xproc_ref[...] = jnp.where(mask, xp, 0.0)


def _run_scan(sf, cw, sigb, sigc, A, woT, bout):
    full = lambda shape: pl.BlockSpec(shape, lambda b: (0,) * len(shape))
    colblk = lambda r: pl.BlockSpec((r, EDP), lambda b: (0, b))
    return pl.pallas_call(
        _scan_body,
        grid=(B_SZ,),
        in_specs=[
            colblk(K_PAD), colblk(4), colblk(DS), colblk(DS),
            full((DS, DS)), full((EDP, DIM)), full((1, DIM)),
        ],
        out_specs=pl.BlockSpec((K_PAD, DIM), lambda b: (b, 0)),
        out_shape=jax.ShapeDtypeStruct((B_SZ * K_PAD, DIM), _F32),
        scratch_shapes=[
            pltpu.VMEM((K_PAD, EDP), _F32),
            pltpu.VMEM((K_PAD, EDP), _F32),
        ],
    )(sf, cw, sigb, sigc, A, woT, bout)


def _sc_inverse_gather(xproc, inv):
    mesh = plsc.VectorSubcoreMesh(core_axis_name="c", subcore_axis_name="s")
    rows_w = BL // NW
    sub = 64

    @functools.partial(
        pl.kernel,
        out_type=jax.ShapeDtypeStruct((BL, DIM), _F32),
        mesh=mesh,
        scratch_types=[
            pltpu.VMEM((sub,), jnp.int32),
            pltpu.VMEM((sub, DIM), _F32),
            pltpu.SemaphoreType.DMA,
        ],
    )
    def inv_k(xproc_hbm, inv_hbm, out_hbm, idx_v, rows_v, sem):
        wid = lax.axis_index("s") * 2 + lax.axis_index("c")

        def body(i, carry):
            base = wid * rows_w + i * sub
            pltpu.sync_copy(inv_hbm.at[pl.ds(base, sub)], idx_v)
            pltpu.async_copy(xproc_hbm.at[idx_v], rows_v, sem).wait()
            pltpu.sync_copy(rows_v, out_hbm.at[pl.ds(base, sub)])
            return carry

        lax.fori_loop(0, rows_w // sub, body, 0)

    return inv_k(xproc, inv)


def _resid_body(x_ref, g_ref, o_ref):
    o_ref[...] = x_ref[...] + g_ref[...]


def _run_resid(x2d, g):
    ntiles = 16
    rows = BL // ntiles
    spec = pl.BlockSpec((rows, DIM), lambda i: (i, 0))
    return pl.pallas_call(
        _resid_body,
        grid=(ntiles,),
        in_specs=[spec, spec],
        out_specs=spec,
        out_shape=jax.ShapeDtypeStruct((BL, DIM), _F32),
    )(x2d, g)


def _head_pad_cols(w):
    n = w.shape[0]
    w3 = w.reshape(n, NH, HD)
    w3 = jnp.pad(w3, ((0, 0), (0, 0), (0, HDP - HD)))
    return w3.reshape(n, EDP)


def _head_pad_vec(b):
    return _head_pad_cols(b.reshape(1, ED))


@jax.jit
def kernel(x, alpha, norm_w, norm_b, W_in, b_in, qkv_W, qkv_b, conv_w,
           A, Bp, Cp, W_out, b_out):
    x2d = x.reshape(BL, DIM)

    winT = jnp.pad(W_in.T, ((0, 0), (0, EDP - ED)))
    bin_p = jnp.pad(b_in, (0, EDP - ED)).reshape(1, EDP)
    wq, wk, wv = qkv_W[0:ED], qkv_W[ED:2 * ED], qkv_W[2 * ED:3 * ED]
    wqT = jnp.pad(_head_pad_cols(wq.T.reshape(ED, ED)), ((0, EDP - ED), (0, 0)))
    wkT = jnp.pad(_head_pad_cols(wk.T), ((0, EDP - ED), (0, 0)))
    wvT = jnp.pad(_head_pad_cols(wv.T), ((0, EDP - ED), (0, 0)))
    bq = _head_pad_vec(qkv_b[0:ED])
    bk = _head_pad_vec(qkv_b[ED:2 * ED])
    bv = _head_pad_vec(qkv_b[2 * ED:3 * ED])
    alpha11 = alpha.reshape(1, 1)
    nw = norm_w.reshape(1, DIM)
    nb = norm_b.reshape(1, DIM)

    cw = _head_pad_cols(conv_w[:, 0, :].T)
    cw2 = jnp.concatenate([cw, cw], axis=1)
    sigb = jnp.broadcast_to(jax.nn.sigmoid(Bp).reshape(DS, 1), (DS, ED))
    sigb = jnp.concatenate([_head_pad_cols(sigb)] * B_SZ, axis=1)
    sigc = jnp.concatenate([_head_pad_cols(jax.nn.sigmoid(Cp).T)] * B_SZ,
                           axis=1)
    woT = _head_pad_cols(W_out).T
    bout = b_out.reshape(1, DIM)

    out, ch2 = _run_front(x2d, winT, bin_p, wqT, wkT, wvT, bq, bk, bv,
                          alpha11, nw, nb)

    _, topk_idx = lax.top_k(ch2, K_CH)
    boff = jnp.arange(B_SZ, dtype=jnp.int32)[:, None] * L
    tpad = jnp.pad(topk_idx.astype(jnp.int32), ((0, 0), (0, K_PAD - K_CH)))
    idx_g = (tpad + boff).reshape(NW * ROWS_W)
    sel_flat = (topk_idx.astype(jnp.int32) + boff).reshape(-1)
    src_flat = (jnp.arange(K_CH, dtype=jnp.int32)[None, :]
                + jnp.arange(B_SZ, dtype=jnp.int32)[:, None] * K_PAD
                ).reshape(-1)
    zrow = jnp.int32(K_CH)
    inv = jnp.full((BL,), zrow, jnp.int32).at[sel_flat].set(src_flat)

    sf = _sc_gather(out, idx_g)

    xproc = _run_scan(sf, cw2, sigb, sigc, A, woT, bout)

    g = _sc_inverse_gather(xproc, inv)
    res = _run_resid(x2d, g)
    return res.reshape(B_SZ, L, DIM)


# speedup vs baseline: 5.9566x; 1.0029x over previous
import functools

import jax
import jax.numpy as jnp
from jax import lax
from jax.experimental import pallas as pl
from jax.experimental.pallas import tpu as pltpu
from jax.experimental.pallas import tpu_sc as plsc

B_SZ = 2
L = 4096
BL = B_SZ * L
DIM = 1024
ED = 336
NH = 6
HD = 56
HDP = 64
EDP = NH * HDP
DS = 16
K_CH = 819
K_PAD = 896
ROWS_W = 56
NW = 32

_F32 = jnp.float32


def _front_body(x_ref, winT_ref, bin_ref, wqT_ref, wkT_ref, wvT_ref,
                bq_ref, bk_ref, bv_ref, alpha_ref, nw_ref, nb_ref,
                out_ref, ch2_ref):
    x = x_ref[...]
    xn = jnp.tanh(x * alpha_ref[...]) * nw_ref[...] + nb_ref[...]
    xp = jnp.dot(xn, winT_ref[...], preferred_element_type=_F32) + bin_ref[...]
    n2 = jnp.sum(xp * xp, axis=1, keepdims=True)
    xp = xp / jnp.maximum(jnp.sqrt(n2), 1e-12)
    q = jnp.dot(xp, wqT_ref[...], preferred_element_type=_F32) + bq_ref[...]
    k = jnp.dot(xp, wkT_ref[...], preferred_element_type=_F32) + bk_ref[...]
    v = jnp.dot(xp, wvT_ref[...], preferred_element_type=_F32) + bv_ref[...]
    outs = []
    for h in range(NH):
        qh = q[:, HDP * h:HDP * (h + 1)]
        best_a = jnp.sum(qh * k[:, 0:HDP], axis=1, keepdims=True)
        out_h = v[:, 0:HDP]
        for j in range(1, NH):
            kj = k[:, HDP * j:HDP * (j + 1)]
            a = jnp.sum(qh * kj, axis=1, keepdims=True)
            upd = a > best_a
            best_a = jnp.where(upd, a, best_a)
            out_h = jnp.where(upd, v[:, HDP * j:HDP * (j + 1)], out_h)
        outs.append(out_h)
    out = jnp.concatenate(outs, axis=1)
    out_ref[...] = out
    ch2_ref[...] = jnp.sum(out * out, axis=1)[None, None, :]


def _run_front(x2d, winT, bin_p, wqT, wkT, wvT, bq, bk, bv, alpha11, nw, nb):
    ntiles = 16
    rows = BL // ntiles
    full = lambda shape: pl.BlockSpec(shape, lambda i: (0,) * len(shape))
    out, ch2 = pl.pallas_call(
        _front_body,
        grid=(ntiles,),
        in_specs=[
            pl.BlockSpec((rows, DIM), lambda i: (i, 0)),
            full((DIM, EDP)), full((1, EDP)),
            full((EDP, EDP)), full((EDP, EDP)), full((EDP, EDP)),
            full((1, EDP)), full((1, EDP)), full((1, EDP)),
            full((1, 1)), full((1, DIM)), full((1, DIM)),
        ],
        out_specs=[
            pl.BlockSpec((rows, EDP), lambda i: (i, 0)),
            pl.BlockSpec((1, 1, rows), lambda i: (i, 0, 0)),
        ],
        out_shape=[
            jax.ShapeDtypeStruct((BL, EDP), _F32),
            jax.ShapeDtypeStruct((ntiles, 1, rows), _F32),
        ],
    )(x2d, winT, bin_p, wqT, wkT, wvT, bq, bk, bv, alpha11, nw, nb)
    return out, ch2.reshape(B_SZ, L)


def _sc_gather(table, idx):
    mesh = plsc.VectorSubcoreMesh(core_axis_name="c", subcore_axis_name="s")

    @functools.partial(
        pl.kernel,
        out_type=jax.ShapeDtypeStruct((K_PAD, B_SZ * EDP), _F32),
        mesh=mesh,
        scratch_types=[
            pltpu.VMEM((ROWS_W,), jnp.int32),
            pltpu.VMEM((ROWS_W, EDP), _F32),
            pltpu.SemaphoreType.DMA,
        ],
    )
    def gather_k(table_hbm, idx_hbm, out_hbm, idx_v, rows_v, sem):
        wid = lax.axis_index("s") * 2 + lax.axis_index("c")
        b = wid // 16
        t0 = (wid % 16) * ROWS_W
        pltpu.sync_copy(idx_hbm.at[pl.ds(wid * ROWS_W, ROWS_W)], idx_v)
        pltpu.async_copy(table_hbm.at[idx_v], rows_v, sem).wait()
        pltpu.sync_copy(rows_v,
                        out_hbm.at[pl.ds(t0, ROWS_W), pl.ds(b * EDP, EDP)])

    return gather_k(table, idx)


def _scan_body(u_ref, cw_ref, sigb_ref, sigc_ref, a_ref, woT_ref, bout_ref,
               xproc_ref, xc_scr, y_scr):
    u = u_ref[...]
    w = cw_ref[...]
    z1 = jnp.zeros((1, EDP), _F32)
    z2 = jnp.zeros((2, EDP), _F32)
    z3 = jnp.zeros((3, EDP), _F32)
    xc = u * w[3:4, :]
    xc = xc + jnp.concatenate([z1, u[:-1, :]], axis=0) * w[2:3, :]
    xc = xc + jnp.concatenate([z2, u[:-2, :]], axis=0) * w[1:2, :]
    xc = xc + jnp.concatenate([z3, u[:-3, :]], axis=0) * w[0:1, :]
    xc_scr[...] = xc
    A = a_ref[...]
    sigb = sigb_ref[...]
    sigc = sigc_ref[...]

    def step(t, h):
        ut = xc_scr[pl.ds(t, 1), :]
        h = jnp.dot(A, h, preferred_element_type=_F32) + sigb * ut
        y_scr[pl.ds(t, 1), :] = jnp.sum(h * sigc, axis=0, keepdims=True)
        return h

    lax.fori_loop(0, K_CH, step, jnp.zeros((DS, EDP), _F32))
    y = y_scr[...]
    xp = jnp.dot(y, woT_ref[...], preferred_element_type=_F32) + bout_ref[...]
    mask = lax.broadcasted_iota(jnp.int32, (K_PAD, 1), 0) < K_CH
    xproc_ref[...] = jnp.where(mask, xp, 0.0)


def _run_scan(sf, cw, sigb, sigc, A, woT, bout):
    full = lambda shape: pl.BlockSpec(shape, lambda b: (0,) * len(shape))
    colblk = lambda r: pl.BlockSpec((r, EDP), lambda b: (0, b))
    return pl.pallas_call(
        _scan_body,
        grid=(B_SZ,),
        in_specs=[
            colblk(K_PAD), colblk(4), colblk(DS), colblk(DS),
            full((DS, DS)), full((EDP, DIM)), full((1, DIM)),
        ],
        out_specs=pl.BlockSpec((K_PAD, DIM), lambda b: (b, 0)),
        out_shape=jax.ShapeDtypeStruct((B_SZ * K_PAD, DIM), _F32),
        scratch_shapes=[
            pltpu.VMEM((K_PAD, EDP), _F32),
            pltpu.VMEM((K_PAD, EDP), _F32),
        ],
    )(sf, cw, sigb, sigc, A, woT, bout)


def _sc_inverse_gather(xproc, inv, x2d):
    mesh = plsc.VectorSubcoreMesh(core_axis_name="c", subcore_axis_name="s")
    rows_w = BL // NW
    sub = 16
    nchunk = rows_w // sub
    nbuf = 2
    vpr = DIM // 16

    @functools.partial(
        pl.kernel,
        out_type=jax.ShapeDtypeStruct((BL, DIM), _F32),
        mesh=mesh,
        scratch_types=[
            pltpu.VMEM((nbuf, sub), jnp.int32),
            pltpu.VMEM((nbuf, sub, DIM), _F32),
            pltpu.VMEM((nbuf, sub, DIM), _F32),
            [pltpu.SemaphoreType.DMA] * nbuf,
            [pltpu.SemaphoreType.DMA] * nbuf,
            [pltpu.SemaphoreType.DMA] * nbuf,
        ],
    )
    def inv_k(xproc_hbm, inv_hbm, x_hbm, out_hbm, idx_v, gbuf, xbuf,
              gsems, xsems, osems):
        wid = lax.axis_index("s") * 2 + lax.axis_index("c")
        row0 = wid * rows_w

        def start(c, slot):
            base = row0 + c * sub
            pltpu.sync_copy(inv_hbm.at[pl.ds(base, sub)], idx_v.at[slot])
            pltpu.async_copy(xproc_hbm.at[idx_v.at[slot]], gbuf.at[slot],
                             gsems[slot])
            pltpu.async_copy(x_hbm.at[pl.ds(base, sub)], xbuf.at[slot],
                             xsems[slot])

        def finish(c, slot):
            base = row0 + c * sub
            pltpu.make_async_copy(xproc_hbm.at[idx_v.at[slot]], gbuf.at[slot],
                                  gsems[slot]).wait()
            pltpu.make_async_copy(x_hbm.at[pl.ds(base, sub)], xbuf.at[slot],
                                  xsems[slot]).wait()

            def addrow(r, carry):
                def addvec(j, carry2):
                    gbuf[slot, r, pl.ds(j * 16, 16)] = (
                        gbuf[slot, r, pl.ds(j * 16, 16)]
                        + xbuf[slot, r, pl.ds(j * 16, 16)])
                    return carry2
                return lax.fori_loop(0, vpr, addvec, carry)

            lax.fori_loop(0, sub, addrow, 0)
            pltpu.async_copy(gbuf.at[slot], out_hbm.at[pl.ds(base, sub)],
                             osems[slot])

        for s in range(nbuf):
            start(s, s)

        def body(rnd, carry):
            for s in range(nbuf):
                c = rnd * nbuf + s
                finish(c, s)
                nxt = c + nbuf

                @pl.when(nxt < nchunk)
                def _(s=s, nxt=nxt):
                    pltpu.make_async_copy(gbuf.at[s],
                                          out_hbm.at[pl.ds(row0, sub)],
                                          osems[s]).wait()
                    start(nxt, s)
            return carry

        lax.fori_loop(0, nchunk // nbuf, body, 0)
        for s in range(nbuf):
            pltpu.make_async_copy(gbuf.at[s], out_hbm.at[pl.ds(row0, sub)],
                                  osems[s]).wait()

    return inv_k(xproc, inv, x2d)


def _head_pad_cols(w):
    n = w.shape[0]
    w3 = w.reshape(n, NH, HD)
    w3 = jnp.pad(w3, ((0, 0), (0, 0), (0, HDP - HD)))
    return w3.reshape(n, EDP)


def _head_pad_vec(b):
    return _head_pad_cols(b.reshape(1, ED))


@jax.jit
def kernel(x, alpha, norm_w, norm_b, W_in, b_in, qkv_W, qkv_b, conv_w,
           A, Bp, Cp, W_out, b_out):
    x2d = x.reshape(BL, DIM)

    winT = jnp.pad(W_in.T, ((0, 0), (0, EDP - ED)))
    bin_p = jnp.pad(b_in, (0, EDP - ED)).reshape(1, EDP)
    wq, wk, wv = qkv_W[0:ED], qkv_W[ED:2 * ED], qkv_W[2 * ED:3 * ED]
    wqT = jnp.pad(_head_pad_cols(wq.T.reshape(ED, ED)), ((0, EDP - ED), (0, 0)))
    wkT = jnp.pad(_head_pad_cols(wk.T), ((0, EDP - ED), (0, 0)))
    wvT = jnp.pad(_head_pad_cols(wv.T), ((0, EDP - ED), (0, 0)))
    bq = _head_pad_vec(qkv_b[0:ED])
    bk = _head_pad_vec(qkv_b[ED:2 * ED])
    bv = _head_pad_vec(qkv_b[2 * ED:3 * ED])
    alpha11 = alpha.reshape(1, 1)
    nw = norm_w.reshape(1, DIM)
    nb = norm_b.reshape(1, DIM)

    cw = _head_pad_cols(conv_w[:, 0, :].T)
    cw2 = jnp.concatenate([cw, cw], axis=1)
    sigb = jnp.broadcast_to(jax.nn.sigmoid(Bp).reshape(DS, 1), (DS, ED))
    sigb = jnp.concatenate([_head_pad_cols(sigb)] * B_SZ, axis=1)
    sigc = jnp.concatenate([_head_pad_cols(jax.nn.sigmoid(Cp).T)] * B_SZ,
                           axis=1)
    woT = _head_pad_cols(W_out).T
    bout = b_out.reshape(1, DIM)

    out, ch2 = _run_front(x2d, winT, bin_p, wqT, wkT, wvT, bq, bk, bv,
                          alpha11, nw, nb)

    _, topk_idx = lax.top_k(ch2, K_CH)
    boff = jnp.arange(B_SZ, dtype=jnp.int32)[:, None] * L
    tpad = jnp.pad(topk_idx.astype(jnp.int32), ((0, 0), (0, K_PAD - K_CH)))
    idx_g = (tpad + boff).reshape(NW * ROWS_W)
    sel_flat = (topk_idx.astype(jnp.int32) + boff).reshape(-1)
    src_flat = (jnp.arange(K_CH, dtype=jnp.int32)[None, :]
                + jnp.arange(B_SZ, dtype=jnp.int32)[:, None] * K_PAD
                ).reshape(-1)
    zrow = jnp.int32(K_CH)
    inv = jnp.full((BL,), zrow, jnp.int32).at[sel_flat].set(src_flat)

    sf = _sc_gather(out, idx_g)

    xproc = _run_scan(sf, cw2, sigb, sigc, A, woT, bout)

    res = _sc_inverse_gather(xproc, inv, x2d)
    return res.reshape(B_SZ, L, DIM)


# speedup vs baseline: 9.0118x; 1.5129x over previous
import functools

import jax
import jax.numpy as jnp
from jax import lax
from jax.experimental import pallas as pl
from jax.experimental.pallas import tpu as pltpu
from jax.experimental.pallas import tpu_sc as plsc

B_SZ = 2
L = 4096
BL = B_SZ * L
DIM = 1024
ED = 336
NH = 6
HD = 56
HDP = 64
EDP = NH * HDP
DS = 16
K_CH = 819
K_PAD = 896
ROWS_W = 56
NW = 32

_F32 = jnp.float32


def _front_body(x_ref, winT_ref, bin_ref, wqT_ref, wkT_ref, wvT_ref,
                bq_ref, bk_ref, bv_ref, alpha_ref, nw_ref, nb_ref,
                out_ref, ch2_ref):
    x = x_ref[...]
    xn = jnp.tanh(x * alpha_ref[...]) * nw_ref[...] + nb_ref[...]
    xp = jnp.dot(xn, winT_ref[...], preferred_element_type=_F32) + bin_ref[...]
    n2 = jnp.sum(xp * xp, axis=1, keepdims=True)
    xp = xp / jnp.maximum(jnp.sqrt(n2), 1e-12)
    q = jnp.dot(xp, wqT_ref[...], preferred_element_type=_F32) + bq_ref[...]
    k = jnp.dot(xp, wkT_ref[...], preferred_element_type=_F32) + bk_ref[...]
    v = jnp.dot(xp, wvT_ref[...], preferred_element_type=_F32) + bv_ref[...]
    outs = []
    for h in range(NH):
        qh = q[:, HDP * h:HDP * (h + 1)]
        best_a = jnp.sum(qh * k[:, 0:HDP], axis=1, keepdims=True)
        out_h = v[:, 0:HDP]
        for j in range(1, NH):
            kj = k[:, HDP * j:HDP * (j + 1)]
            a = jnp.sum(qh * kj, axis=1, keepdims=True)
            upd = a > best_a
            best_a = jnp.where(upd, a, best_a)
            out_h = jnp.where(upd, v[:, HDP * j:HDP * (j + 1)], out_h)
        outs.append(out_h)
    out = jnp.concatenate(outs, axis=1)
    out_ref[...] = out
    ch2_ref[...] = jnp.sum(out * out, axis=1)[None, None, :]


def _run_front(x2d, winT, bin_p, wqT, wkT, wvT, bq, bk, bv, alpha11, nw, nb):
    ntiles = 16
    rows = BL // ntiles
    full = lambda shape: pl.BlockSpec(shape, lambda i: (0,) * len(shape))
    out, ch2 = pl.pallas_call(
        _front_body,
        grid=(ntiles,),
        in_specs=[
            pl.BlockSpec((rows, DIM), lambda i: (i, 0)),
            full((DIM, EDP)), full((1, EDP)),
            full((EDP, EDP)), full((EDP, EDP)), full((EDP, EDP)),
            full((1, EDP)), full((1, EDP)), full((1, EDP)),
            full((1, 1)), full((1, DIM)), full((1, DIM)),
        ],
        out_specs=[
            pl.BlockSpec((rows, EDP), lambda i: (i, 0)),
            pl.BlockSpec((1, 1, rows), lambda i: (i, 0, 0)),
        ],
        out_shape=[
            jax.ShapeDtypeStruct((BL, EDP), _F32),
            jax.ShapeDtypeStruct((ntiles, 1, rows), _F32),
        ],
    )(x2d, winT, bin_p, wqT, wkT, wvT, bq, bk, bv, alpha11, nw, nb)
    return out, ch2.reshape(B_SZ, L)


def _sc_gather(table, x2d, idx):
    mesh = plsc.VectorSubcoreMesh(core_axis_name="c", subcore_axis_name="s")

    @functools.partial(
        pl.kernel,
        out_type=[
            jax.ShapeDtypeStruct((K_PAD, B_SZ * EDP), _F32),
            jax.ShapeDtypeStruct((NW * ROWS_W, DIM), _F32),
        ],
        mesh=mesh,
        scratch_types=[
            pltpu.VMEM((ROWS_W,), jnp.int32),
            pltpu.VMEM((ROWS_W, EDP), _F32),
            pltpu.VMEM((ROWS_W, DIM), _F32),
            pltpu.SemaphoreType.DMA,
            pltpu.SemaphoreType.DMA,
        ],
    )
    def gather_k(table_hbm, x_hbm, idx_hbm, out_hbm, xres_hbm,
                 idx_v, rows_v, xrow_v, sem, sem2):
        wid = lax.axis_index("s") * 2 + lax.axis_index("c")
        b = wid // 16
        t0 = (wid % 16) * ROWS_W
        pltpu.sync_copy(idx_hbm.at[pl.ds(wid * ROWS_W, ROWS_W)], idx_v)
        pltpu.async_copy(table_hbm.at[idx_v], rows_v, sem)
        pltpu.async_copy(x_hbm.at[idx_v], xrow_v, sem2)
        pltpu.make_async_copy(table_hbm.at[idx_v], rows_v, sem).wait()
        pltpu.make_async_copy(x_hbm.at[idx_v], xrow_v, sem2).wait()
        pltpu.sync_copy(rows_v,
                        out_hbm.at[pl.ds(t0, ROWS_W), pl.ds(b * EDP, EDP)])
        pltpu.sync_copy(xrow_v, xres_hbm.at[pl.ds(wid * ROWS_W, ROWS_W)])

    return gather_k(table, x2d, idx)


def _scan_body(u_ref, xres_ref, cw_ref, sigb_ref, sigc_ref, a_ref, woT_ref,
               bout_ref, xproc_ref, xc_scr, y_scr):
    u = u_ref[...]
    w = cw_ref[...]
    z1 = jnp.zeros((1, EDP), _F32)
    z2 = jnp.zeros((2, EDP), _F32)
    z3 = jnp.zeros((3, EDP), _F32)
    xc = u * w[3:4, :]
    xc = xc + jnp.concatenate([z1, u[:-1, :]], axis=0) * w[2:3, :]
    xc = xc + jnp.concatenate([z2, u[:-2, :]], axis=0) * w[1:2, :]
    xc = xc + jnp.concatenate([z3, u[:-3, :]], axis=0) * w[0:1, :]
    xc_scr[...] = xc
    A = a_ref[...]
    sigb = sigb_ref[...]
    sigc = sigc_ref[...]

    def step(t, h):
        ut = xc_scr[pl.ds(t, 1), :]
        h = jnp.dot(A, h, preferred_element_type=_F32) + sigb * ut
        y_scr[pl.ds(t, 1), :] = jnp.sum(h * sigc, axis=0, keepdims=True)
        return h

    lax.fori_loop(0, K_CH, step, jnp.zeros((DS, EDP), _F32))
    y = y_scr[...]
    xp = jnp.dot(y, woT_ref[...], preferred_element_type=_F32) + bout_ref[...]
    xproc_ref[...] = xp + xres_ref[...]


def _run_scan(sf, xres, cw, sigb, sigc, A, woT, bout):
    full = lambda shape: pl.BlockSpec(shape, lambda b: (0,) * len(shape))
    colblk = lambda r: pl.BlockSpec((r, EDP), lambda b: (0, b))
    return pl.pallas_call(
        _scan_body,
        grid=(B_SZ,),
        in_specs=[
            colblk(K_PAD),
            pl.BlockSpec((K_PAD, DIM), lambda b: (b, 0)),
            colblk(4), colblk(DS), colblk(DS),
            full((DS, DS)), full((EDP, DIM)), full((1, DIM)),
        ],
        out_specs=pl.BlockSpec((K_PAD, DIM), lambda b: (b, 0)),
        out_shape=jax.ShapeDtypeStruct((B_SZ * K_PAD, DIM), _F32),
        scratch_shapes=[
            pltpu.VMEM((K_PAD, EDP), _F32),
            pltpu.VMEM((K_PAD, EDP), _F32),
        ],
    )(sf, xres, cw, sigb, sigc, A, woT, bout)


BLT = BL + 128


def _sc_scatter(xsum, idx_sc, x2d):
    mesh = plsc.VectorSubcoreMesh(core_axis_name="c", subcore_axis_name="s")
    rows_w = L // 16
    srows = (NW * ROWS_W) // 16

    @functools.partial(
        pl.kernel,
        out_type=jax.ShapeDtypeStruct((BLT, DIM), _F32),
        mesh=mesh,
        scratch_types=[
            pltpu.VMEM((srows,), jnp.int32),
            pltpu.VMEM((srows, DIM), _F32),
            pltpu.SemaphoreType.DMA,
        ],
    )
    def scat_k(xsum_hbm, idx_hbm, x_hbm, out_hbm, idx_v, buf, sem):
        c = lax.axis_index("c")
        s = lax.axis_index("s")
        base = c * L + s * rows_w
        for off, n in ((0, 112), (112, 112), (224, 32)):
            pltpu.sync_copy(x_hbm.at[pl.ds(base + off, n)],
                            buf.at[pl.ds(0, n)])
            pltpu.sync_copy(buf.at[pl.ds(0, n)],
                            out_hbm.at[pl.ds(base + off, n)])
        plsc.subcore_barrier()
        pltpu.sync_copy(idx_hbm.at[c, s], idx_v)
        pltpu.sync_copy(xsum_hbm.at[pl.ds(s * srows, srows)], buf)
        pltpu.async_copy(buf, out_hbm.at[idx_v], sem).wait()

    return scat_k(xsum, idx_sc, x2d)


def _head_pad_cols(w):
    n = w.shape[0]
    w3 = w.reshape(n, NH, HD)
    w3 = jnp.pad(w3, ((0, 0), (0, 0), (0, HDP - HD)))
    return w3.reshape(n, EDP)


def _head_pad_vec(b):
    return _head_pad_cols(b.reshape(1, ED))


@jax.jit
def kernel(x, alpha, norm_w, norm_b, W_in, b_in, qkv_W, qkv_b, conv_w,
           A, Bp, Cp, W_out, b_out):
    x2d = x.reshape(BL, DIM)

    winT = jnp.pad(W_in.T, ((0, 0), (0, EDP - ED)))
    bin_p = jnp.pad(b_in, (0, EDP - ED)).reshape(1, EDP)
    wq, wk, wv = qkv_W[0:ED], qkv_W[ED:2 * ED], qkv_W[2 * ED:3 * ED]
    wqT = jnp.pad(_head_pad_cols(wq.T.reshape(ED, ED)), ((0, EDP - ED), (0, 0)))
    wkT = jnp.pad(_head_pad_cols(wk.T), ((0, EDP - ED), (0, 0)))
    wvT = jnp.pad(_head_pad_cols(wv.T), ((0, EDP - ED), (0, 0)))
    bq = _head_pad_vec(qkv_b[0:ED])
    bk = _head_pad_vec(qkv_b[ED:2 * ED])
    bv = _head_pad_vec(qkv_b[2 * ED:3 * ED])
    alpha11 = alpha.reshape(1, 1)
    nw = norm_w.reshape(1, DIM)
    nb = norm_b.reshape(1, DIM)

    cw = _head_pad_cols(conv_w[:, 0, :].T)
    cw2 = jnp.concatenate([cw, cw], axis=1)
    sigb = jnp.broadcast_to(jax.nn.sigmoid(Bp).reshape(DS, 1), (DS, ED))
    sigb = jnp.concatenate([_head_pad_cols(sigb)] * B_SZ, axis=1)
    sigc = jnp.concatenate([_head_pad_cols(jax.nn.sigmoid(Cp).T)] * B_SZ,
                           axis=1)
    woT = _head_pad_cols(W_out).T
    bout = b_out.reshape(1, DIM)

    out, ch2 = _run_front(x2d, winT, bin_p, wqT, wkT, wvT, bq, bk, bv,
                          alpha11, nw, nb)

    _, topk_idx = lax.top_k(ch2, K_CH)
    boff = jnp.arange(B_SZ, dtype=jnp.int32)[:, None] * L
    tpad = jnp.pad(topk_idx.astype(jnp.int32), ((0, 0), (0, K_PAD - K_CH)))
    tgt = tpad + boff
    valid = (jnp.arange(K_PAD, dtype=jnp.int32) < K_CH)[None, :]
    trash = BL + (jnp.arange(K_PAD, dtype=jnp.int32) % 128)[None, :]
    tgt = jnp.where(valid, tgt, trash)
    idx_g = (tpad + boff).reshape(NW * ROWS_W)
    halves = jnp.arange(2, dtype=jnp.int32)[:, None] * L
    tflat = tgt.reshape(1, -1)
    trash_flat = BL + (jnp.arange(NW * ROWS_W, dtype=jnp.int32) % 128)[None, :]
    own = (tflat >= halves) & (tflat < halves + L)
    idx_sc = jnp.where(own, tflat, trash_flat)
    idx_sc = idx_sc.reshape(B_SZ, 16, (NW * ROWS_W) // 16)

    sf, xres = _sc_gather(out, x2d, idx_g)

    xsum = _run_scan(sf, xres, cw2, sigb, sigc, A, woT, bout)

    res = _sc_scatter(xsum, idx_sc, x2d)
    return res[:BL].reshape(B_SZ, L, DIM)


# speedup vs baseline: 13.7722x; 1.5282x over previous
import functools

import jax
import jax.numpy as jnp
from jax import lax
from jax.experimental import pallas as pl
from jax.experimental.pallas import tpu as pltpu
from jax.experimental.pallas import tpu_sc as plsc

B_SZ = 2
L = 4096
BL = B_SZ * L
DIM = 1024
ED = 336
NH = 6
HD = 56
HDP = 64
EDP = NH * HDP
DS = 16
K_CH = 819
K_PAD = 896
ROWS_W = 56
NW = 32

_F32 = jnp.float32


def _front_body(x_ref, winT_ref, bin_ref, wqT_ref, wkT_ref, wvT_ref,
                bq_ref, bk_ref, bv_ref, alpha_ref, nw_ref, nb_ref,
                out_ref, ch2_ref):
    x = x_ref[...]
    xn = jnp.tanh(x * alpha_ref[...]) * nw_ref[...] + nb_ref[...]
    xp = jnp.dot(xn, winT_ref[...], preferred_element_type=_F32) + bin_ref[...]
    ones8 = jnp.ones((EDP, 8), _F32)
    hrow = lax.broadcasted_iota(jnp.int32, (EDP, 8), 0) // HDP
    hcol = lax.broadcasted_iota(jnp.int32, (EDP, 8), 1)
    S = (hrow == hcol).astype(_F32)
    n2 = jnp.dot(xp * xp, ones8, preferred_element_type=_F32)[:, 0:1]
    xp = xp / jnp.maximum(jnp.sqrt(n2), 1e-12)
    q = jnp.dot(xp, wqT_ref[...], preferred_element_type=_F32) + bq_ref[...]
    k = jnp.dot(xp, wkT_ref[...], preferred_element_type=_F32) + bk_ref[...]
    v = jnp.dot(xp, wvT_ref[...], preferred_element_type=_F32) + bv_ref[...]
    a_js = []
    for j in range(NH):
        kj = k[:, HDP * j:HDP * (j + 1)]
        kt = jnp.concatenate([kj] * NH, axis=1)
        a_js.append(jnp.dot(q * kt, S, preferred_element_type=_F32))
    best_a = a_js[0]
    upds = []
    for j in range(1, NH):
        upd = a_js[j] > best_a
        best_a = jnp.where(upd, a_js[j], best_a)
        upds.append(upd)
    outs = []
    for h in range(NH):
        out_h = v[:, 0:HDP]
        for j in range(1, NH):
            out_h = jnp.where(upds[j - 1][:, h:h + 1],
                              v[:, HDP * j:HDP * (j + 1)], out_h)
        outs.append(out_h)
    out = jnp.concatenate(outs, axis=1)
    out_ref[...] = out
    ch2 = jnp.dot(out * out, ones8, preferred_element_type=_F32)[:, 0]
    ch2_ref[...] = ch2[None, None, :]


def _run_front(x2d, winT, bin_p, wqT, wkT, wvT, bq, bk, bv, alpha11, nw, nb):
    ntiles = 16
    rows = BL // ntiles
    full = lambda shape: pl.BlockSpec(shape, lambda i: (0,) * len(shape))
    out, ch2 = pl.pallas_call(
        _front_body,
        grid=(ntiles,),
        in_specs=[
            pl.BlockSpec((rows, DIM), lambda i: (i, 0)),
            full((DIM, EDP)), full((1, EDP)),
            full((EDP, EDP)), full((EDP, EDP)), full((EDP, EDP)),
            full((1, EDP)), full((1, EDP)), full((1, EDP)),
            full((1, 1)), full((1, DIM)), full((1, DIM)),
        ],
        out_specs=[
            pl.BlockSpec((rows, EDP), lambda i: (i, 0)),
            pl.BlockSpec((1, 1, rows), lambda i: (i, 0, 0)),
        ],
        out_shape=[
            jax.ShapeDtypeStruct((BL, EDP), _F32),
            jax.ShapeDtypeStruct((ntiles, 1, rows), _F32),
        ],
    )(x2d, winT, bin_p, wqT, wkT, wvT, bq, bk, bv, alpha11, nw, nb)
    return out, ch2.reshape(B_SZ, L)


def _sc_gather(table, x2d, idx):
    mesh = plsc.VectorSubcoreMesh(core_axis_name="c", subcore_axis_name="s")

    @functools.partial(
        pl.kernel,
        out_type=[
            jax.ShapeDtypeStruct((K_PAD, B_SZ * EDP), _F32),
            jax.ShapeDtypeStruct((NW * ROWS_W, DIM), _F32),
        ],
        mesh=mesh,
        scratch_types=[
            pltpu.VMEM((ROWS_W,), jnp.int32),
            pltpu.VMEM((ROWS_W, EDP), _F32),
            pltpu.VMEM((ROWS_W, DIM), _F32),
            pltpu.SemaphoreType.DMA,
            pltpu.SemaphoreType.DMA,
        ],
    )
    def gather_k(table_hbm, x_hbm, idx_hbm, out_hbm, xres_hbm,
                 idx_v, rows_v, xrow_v, sem, sem2):
        wid = lax.axis_index("s") * 2 + lax.axis_index("c")
        b = wid // 16
        t0 = (wid % 16) * ROWS_W
        pltpu.sync_copy(idx_hbm.at[pl.ds(wid * ROWS_W, ROWS_W)], idx_v)
        pltpu.async_copy(table_hbm.at[idx_v], rows_v, sem)
        pltpu.async_copy(x_hbm.at[idx_v], xrow_v, sem2)
        pltpu.make_async_copy(table_hbm.at[idx_v], rows_v, sem).wait()
        pltpu.make_async_copy(x_hbm.at[idx_v], xrow_v, sem2).wait()
        pltpu.sync_copy(rows_v,
                        out_hbm.at[pl.ds(t0, ROWS_W), pl.ds(b * EDP, EDP)])
        pltpu.sync_copy(xrow_v, xres_hbm.at[pl.ds(wid * ROWS_W, ROWS_W)])

    return gather_k(table, x2d, idx)


_W2 = B_SZ * EDP


def _scan_body(u_ref, xres_ref, cw_ref, sigb_ref, sigc_ref, a_ref, woT_ref,
               bout_ref, xproc_ref, xc_scr, y_scr):
    u = u_ref[...]
    w = cw_ref[...]
    z1 = jnp.zeros((1, _W2), _F32)
    z2 = jnp.zeros((2, _W2), _F32)
    z3 = jnp.zeros((3, _W2), _F32)
    xc = u * w[3:4, :]
    xc = xc + jnp.concatenate([z1, u[:-1, :]], axis=0) * w[2:3, :]
    xc = xc + jnp.concatenate([z2, u[:-2, :]], axis=0) * w[1:2, :]
    xc = xc + jnp.concatenate([z3, u[:-3, :]], axis=0) * w[0:1, :]
    xc_scr[...] = xc
    A = a_ref[...]
    sigb = sigb_ref[...]
    sigc = sigc_ref[...]

    def step(t, h):
        ut = xc_scr[pl.ds(t, 1), :]
        h = jnp.dot(A, h, preferred_element_type=_F32) + sigb * ut
        y_scr[pl.ds(t, 1), :] = jnp.sum(h * sigc, axis=0, keepdims=True)
        return h

    lax.fori_loop(0, K_CH, step, jnp.zeros((DS, _W2), _F32), unroll=4)
    y = y_scr[...]
    woT = woT_ref[...]
    bout = bout_ref[...]
    for b in range(B_SZ):
        yb = y[:, b * EDP:(b + 1) * EDP]
        xp = jnp.dot(yb, woT, preferred_element_type=_F32) + bout
        xproc_ref[pl.ds(b * K_PAD, K_PAD), :] = (
            xp + xres_ref[pl.ds(b * K_PAD, K_PAD), :])


def _run_scan(sf, xres, cw, sigb, sigc, A, woT, bout):
    full = lambda shape: pl.BlockSpec(shape, lambda b: (0,) * len(shape))
    return pl.pallas_call(
        _scan_body,
        grid=(1,),
        in_specs=[
            full((K_PAD, _W2)),
            full((B_SZ * K_PAD, DIM)),
            full((4, _W2)), full((DS, _W2)), full((DS, _W2)),
            full((DS, DS)), full((EDP, DIM)), full((1, DIM)),
        ],
        out_specs=full((B_SZ * K_PAD, DIM)),
        out_shape=jax.ShapeDtypeStruct((B_SZ * K_PAD, DIM), _F32),
        scratch_shapes=[
            pltpu.VMEM((K_PAD, _W2), _F32),
            pltpu.VMEM((K_PAD, _W2), _F32),
        ],
    )(sf, xres, cw, sigb, sigc, A, woT, bout)


BLT = BL + 128


def _sc_scatter(xsum, idx_sc, x2d):
    mesh = plsc.VectorSubcoreMesh(core_axis_name="c", subcore_axis_name="s")
    rows_w = L // 16
    srows = (NW * ROWS_W) // 16

    @functools.partial(
        pl.kernel,
        out_type=jax.ShapeDtypeStruct((BLT, DIM), _F32),
        mesh=mesh,
        scratch_types=[
            pltpu.VMEM((srows,), jnp.int32),
            pltpu.VMEM((srows, DIM), _F32),
            pltpu.SemaphoreType.DMA,
        ],
    )
    def scat_k(xsum_hbm, idx_hbm, x_hbm, out_hbm, idx_v, buf, sem):
        c = lax.axis_index("c")
        s = lax.axis_index("s")
        base = c * L + s * rows_w
        for off, n in ((0, 112), (112, 112), (224, 32)):
            pltpu.sync_copy(x_hbm.at[pl.ds(base + off, n)],
                            buf.at[pl.ds(0, n)])
            pltpu.sync_copy(buf.at[pl.ds(0, n)],
                            out_hbm.at[pl.ds(base + off, n)])
        plsc.subcore_barrier()
        pltpu.sync_copy(idx_hbm.at[c, s], idx_v)
        pltpu.sync_copy(xsum_hbm.at[pl.ds(s * srows, srows)], buf)
        pltpu.async_copy(buf, out_hbm.at[idx_v], sem).wait()

    return scat_k(xsum, idx_sc, x2d)


def _head_pad_cols(w):
    n = w.shape[0]
    w3 = w.reshape(n, NH, HD)
    w3 = jnp.pad(w3, ((0, 0), (0, 0), (0, HDP - HD)))
    return w3.reshape(n, EDP)


def _head_pad_vec(b):
    return _head_pad_cols(b.reshape(1, ED))


@jax.jit
def kernel(x, alpha, norm_w, norm_b, W_in, b_in, qkv_W, qkv_b, conv_w,
           A, Bp, Cp, W_out, b_out):
    x2d = x.reshape(BL, DIM)

    winT = jnp.pad(W_in.T, ((0, 0), (0, EDP - ED)))
    bin_p = jnp.pad(b_in, (0, EDP - ED)).reshape(1, EDP)
    wq, wk, wv = qkv_W[0:ED], qkv_W[ED:2 * ED], qkv_W[2 * ED:3 * ED]
    wqT = jnp.pad(_head_pad_cols(wq.T.reshape(ED, ED)), ((0, EDP - ED), (0, 0)))
    wkT = jnp.pad(_head_pad_cols(wk.T), ((0, EDP - ED), (0, 0)))
    wvT = jnp.pad(_head_pad_cols(wv.T), ((0, EDP - ED), (0, 0)))
    bq = _head_pad_vec(qkv_b[0:ED])
    bk = _head_pad_vec(qkv_b[ED:2 * ED])
    bv = _head_pad_vec(qkv_b[2 * ED:3 * ED])
    alpha11 = alpha.reshape(1, 1)
    nw = norm_w.reshape(1, DIM)
    nb = norm_b.reshape(1, DIM)

    cw = _head_pad_cols(conv_w[:, 0, :].T)
    cw2 = jnp.concatenate([cw, cw], axis=1)
    sigb = jnp.broadcast_to(jax.nn.sigmoid(Bp).reshape(DS, 1), (DS, ED))
    sigb = jnp.concatenate([_head_pad_cols(sigb)] * B_SZ, axis=1)
    sigc = jnp.concatenate([_head_pad_cols(jax.nn.sigmoid(Cp).T)] * B_SZ,
                           axis=1)
    woT = _head_pad_cols(W_out).T
    bout = b_out.reshape(1, DIM)

    out, ch2 = _run_front(x2d, winT, bin_p, wqT, wkT, wvT, bq, bk, bv,
                          alpha11, nw, nb)

    _, topk_idx = lax.top_k(ch2, K_CH)
    boff = jnp.arange(B_SZ, dtype=jnp.int32)[:, None] * L
    tpad = jnp.pad(topk_idx.astype(jnp.int32), ((0, 0), (0, K_PAD - K_CH)))
    tgt = tpad + boff
    valid = (jnp.arange(K_PAD, dtype=jnp.int32) < K_CH)[None, :]
    trash = BL + (jnp.arange(K_PAD, dtype=jnp.int32) % 128)[None, :]
    tgt = jnp.where(valid, tgt, trash)
    idx_g = (tpad + boff).reshape(NW * ROWS_W)
    halves = jnp.arange(2, dtype=jnp.int32)[:, None] * L
    tflat = tgt.reshape(1, -1)
    trash_flat = BL + (jnp.arange(NW * ROWS_W, dtype=jnp.int32) % 128)[None, :]
    own = (tflat >= halves) & (tflat < halves + L)
    idx_sc = jnp.where(own, tflat, trash_flat)
    idx_sc = idx_sc.reshape(B_SZ, 16, (NW * ROWS_W) // 16)

    sf, xres = _sc_gather(out, x2d, idx_g)

    xsum = _run_scan(sf, xres, cw2, sigb, sigc, A, woT, bout)

    res = _sc_scatter(xsum, idx_sc, x2d)
    return res[:BL].reshape(B_SZ, L, DIM)


# speedup vs baseline: 13.8167x; 1.0032x over previous
import functools

import jax
import jax.numpy as jnp
from jax import lax
from jax.experimental import pallas as pl
from jax.experimental.pallas import tpu as pltpu
from jax.experimental.pallas import tpu_sc as plsc

B_SZ = 2
L = 4096
BL = B_SZ * L
DIM = 1024
ED = 336
NH = 6
HD = 56
HDP = 64
EDP = NH * HDP
DS = 16
K_CH = 819
K_PAD = 896
ROWS_W = 56
NW = 32

_F32 = jnp.float32


def _front_body(x_ref, winT_ref, bin_ref, wqT_ref, wkT_ref, wvT_ref,
                bq_ref, bk_ref, bv_ref, alpha_ref, nw_ref, nb_ref,
                out_ref, ch2_ref):
    x = x_ref[...]
    xn = jnp.tanh(x * alpha_ref[...]) * nw_ref[...] + nb_ref[...]
    xp = jnp.dot(xn.astype(jnp.bfloat16), winT_ref[...],
                 preferred_element_type=_F32) + bin_ref[...]
    ones8 = jnp.ones((EDP, 8), _F32)
    hrow = lax.broadcasted_iota(jnp.int32, (EDP, 8), 0) // HDP
    hcol = lax.broadcasted_iota(jnp.int32, (EDP, 8), 1)
    S = (hrow == hcol).astype(_F32)
    n2 = jnp.dot(xp * xp, ones8, preferred_element_type=_F32)[:, 0:1]
    xp = xp / jnp.maximum(jnp.sqrt(n2), 1e-12)
    xpb = xp.astype(jnp.bfloat16)
    q = jnp.dot(xpb, wqT_ref[...], preferred_element_type=_F32) + bq_ref[...]
    k = jnp.dot(xpb, wkT_ref[...], preferred_element_type=_F32) + bk_ref[...]
    v = jnp.dot(xpb, wvT_ref[...], preferred_element_type=_F32) + bv_ref[...]
    a_js = []
    for j in range(NH):
        kj = k[:, HDP * j:HDP * (j + 1)]
        kt = jnp.concatenate([kj] * NH, axis=1)
        a_js.append(jnp.dot(q * kt, S, preferred_element_type=_F32))
    best_a = a_js[0]
    upds = []
    for j in range(1, NH):
        upd = a_js[j] > best_a
        best_a = jnp.where(upd, a_js[j], best_a)
        upds.append(upd)
    outs = []
    for h in range(NH):
        out_h = v[:, 0:HDP]
        for j in range(1, NH):
            out_h = jnp.where(upds[j - 1][:, h:h + 1],
                              v[:, HDP * j:HDP * (j + 1)], out_h)
        outs.append(out_h)
    out = jnp.concatenate(outs, axis=1)
    out_ref[...] = out
    ch2 = jnp.dot(out * out, ones8, preferred_element_type=_F32)[:, 0]
    ch2_ref[...] = ch2[None, None, :]


def _run_front(x2d, winT, bin_p, wqT, wkT, wvT, bq, bk, bv, alpha11, nw, nb):
    ntiles = 16
    rows = BL // ntiles
    full = lambda shape: pl.BlockSpec(shape, lambda i: (0,) * len(shape))
    out, ch2 = pl.pallas_call(
        _front_body,
        grid=(ntiles,),
        in_specs=[
            pl.BlockSpec((rows, DIM), lambda i: (i, 0)),
            full((DIM, EDP)), full((1, EDP)),
            full((EDP, EDP)), full((EDP, EDP)), full((EDP, EDP)),
            full((1, EDP)), full((1, EDP)), full((1, EDP)),
            full((1, 1)), full((1, DIM)), full((1, DIM)),
        ],
        out_specs=[
            pl.BlockSpec((rows, EDP), lambda i: (i, 0)),
            pl.BlockSpec((1, 1, rows), lambda i: (i, 0, 0)),
        ],
        out_shape=[
            jax.ShapeDtypeStruct((BL, EDP), _F32),
            jax.ShapeDtypeStruct((ntiles, 1, rows), _F32),
        ],
    )(x2d, winT, bin_p, wqT, wkT, wvT, bq, bk, bv, alpha11, nw, nb)
    return out, ch2.reshape(B_SZ, L)


def _sc_gather(table, x2d, idx):
    mesh = plsc.VectorSubcoreMesh(core_axis_name="c", subcore_axis_name="s")

    @functools.partial(
        pl.kernel,
        out_type=[
            jax.ShapeDtypeStruct((K_PAD, B_SZ * EDP), _F32),
            jax.ShapeDtypeStruct((NW * ROWS_W, DIM), _F32),
        ],
        mesh=mesh,
        scratch_types=[
            pltpu.VMEM((ROWS_W,), jnp.int32),
            pltpu.VMEM((ROWS_W, EDP), _F32),
            pltpu.VMEM((ROWS_W, DIM), _F32),
            pltpu.SemaphoreType.DMA,
            pltpu.SemaphoreType.DMA,
        ],
    )
    def gather_k(table_hbm, x_hbm, idx_hbm, out_hbm, xres_hbm,
                 idx_v, rows_v, xrow_v, sem, sem2):
        wid = lax.axis_index("s") * 2 + lax.axis_index("c")
        b = wid // 16
        t0 = (wid % 16) * ROWS_W
        pltpu.sync_copy(idx_hbm.at[pl.ds(wid * ROWS_W, ROWS_W)], idx_v)
        pltpu.async_copy(table_hbm.at[idx_v], rows_v, sem)
        pltpu.async_copy(x_hbm.at[idx_v], xrow_v, sem2)
        pltpu.make_async_copy(table_hbm.at[idx_v], rows_v, sem).wait()
        pltpu.make_async_copy(x_hbm.at[idx_v], xrow_v, sem2).wait()
        pltpu.sync_copy(rows_v,
                        out_hbm.at[pl.ds(t0, ROWS_W), pl.ds(b * EDP, EDP)])
        pltpu.sync_copy(xrow_v, xres_hbm.at[pl.ds(wid * ROWS_W, ROWS_W)])

    return gather_k(table, x2d, idx)


_W2 = B_SZ * EDP


def _scan_body(u_ref, xres_ref, cw_ref, sigb_ref, sigc_ref, a_ref, woT_ref,
               bout_ref, xproc_ref, xc_scr, y_scr):
    u = u_ref[...]
    w = cw_ref[...]
    z1 = jnp.zeros((1, _W2), _F32)
    z2 = jnp.zeros((2, _W2), _F32)
    z3 = jnp.zeros((3, _W2), _F32)
    xc = u * w[3:4, :]
    xc = xc + jnp.concatenate([z1, u[:-1, :]], axis=0) * w[2:3, :]
    xc = xc + jnp.concatenate([z2, u[:-2, :]], axis=0) * w[1:2, :]
    xc = xc + jnp.concatenate([z3, u[:-3, :]], axis=0) * w[0:1, :]
    xc_scr[...] = xc
    A = a_ref[...]
    sigb = sigb_ref[...]
    sigc = sigc_ref[...]

    def step(t, h):
        ut = xc_scr[pl.ds(t, 1), :]
        h = (jnp.dot(A, h.astype(jnp.bfloat16), preferred_element_type=_F32)
             + sigb * ut)
        y_scr[pl.ds(t, 1), :] = jnp.sum(h * sigc, axis=0, keepdims=True)
        return h

    lax.fori_loop(0, K_CH, step, jnp.zeros((DS, _W2), _F32), unroll=4)
    y = y_scr[...]
    woT = woT_ref[...]
    bout = bout_ref[...]
    for b in range(B_SZ):
        yb = y[:, b * EDP:(b + 1) * EDP].astype(jnp.bfloat16)
        xp = jnp.dot(yb, woT, preferred_element_type=_F32) + bout
        xproc_ref[pl.ds(b * K_PAD, K_PAD), :] = (
            xp + xres_ref[pl.ds(b * K_PAD, K_PAD), :])


def _run_scan(sf, xres, cw, sigb, sigc, A, woT, bout):
    full = lambda shape: pl.BlockSpec(shape, lambda b: (0,) * len(shape))
    return pl.pallas_call(
        _scan_body,
        grid=(1,),
        in_specs=[
            full((K_PAD, _W2)),
            full((B_SZ * K_PAD, DIM)),
            full((4, _W2)), full((DS, _W2)), full((DS, _W2)),
            full((DS, DS)), full((EDP, DIM)), full((1, DIM)),
        ],
        out_specs=full((B_SZ * K_PAD, DIM)),
        out_shape=jax.ShapeDtypeStruct((B_SZ * K_PAD, DIM), _F32),
        scratch_shapes=[
            pltpu.VMEM((K_PAD, _W2), _F32),
            pltpu.VMEM((K_PAD, _W2), _F32),
        ],
    )(sf, xres, cw, sigb, sigc, A, woT, bout)


BLT = BL + 128


def _sc_scatter(xsum, idx_sc, x2d):
    mesh = plsc.VectorSubcoreMesh(core_axis_name="c", subcore_axis_name="s")
    rows_w = L // 16
    srows = (NW * ROWS_W) // 16

    @functools.partial(
        pl.kernel,
        out_type=jax.ShapeDtypeStruct((BLT, DIM), _F32),
        mesh=mesh,
        scratch_types=[
            pltpu.VMEM((srows,), jnp.int32),
            pltpu.VMEM((srows, DIM), _F32),
            pltpu.SemaphoreType.DMA,
        ],
    )
    def scat_k(xsum_hbm, idx_hbm, x_hbm, out_hbm, idx_v, buf, sem):
        c = lax.axis_index("c")
        s = lax.axis_index("s")
        base = c * L + s * rows_w
        for off, n in ((0, 112), (112, 112), (224, 32)):
            pltpu.sync_copy(x_hbm.at[pl.ds(base + off, n)],
                            buf.at[pl.ds(0, n)])
            pltpu.sync_copy(buf.at[pl.ds(0, n)],
                            out_hbm.at[pl.ds(base + off, n)])
        plsc.subcore_barrier()
        pltpu.sync_copy(idx_hbm.at[c, s], idx_v)
        pltpu.sync_copy(xsum_hbm.at[pl.ds(s * srows, srows)], buf)
        pltpu.async_copy(buf, out_hbm.at[idx_v], sem).wait()

    return scat_k(xsum, idx_sc, x2d)


def _head_pad_cols(w):
    n = w.shape[0]
    w3 = w.reshape(n, NH, HD)
    w3 = jnp.pad(w3, ((0, 0), (0, 0), (0, HDP - HD)))
    return w3.reshape(n, EDP)


def _head_pad_vec(b):
    return _head_pad_cols(b.reshape(1, ED))


@jax.jit
def kernel(x, alpha, norm_w, norm_b, W_in, b_in, qkv_W, qkv_b, conv_w,
           A, Bp, Cp, W_out, b_out):
    x2d = x.reshape(BL, DIM)

    bf16 = jnp.bfloat16
    winT = jnp.pad(W_in.T, ((0, 0), (0, EDP - ED))).astype(bf16)
    bin_p = jnp.pad(b_in, (0, EDP - ED)).reshape(1, EDP)
    wq, wk, wv = qkv_W[0:ED], qkv_W[ED:2 * ED], qkv_W[2 * ED:3 * ED]
    wqT = jnp.pad(_head_pad_cols(wq.T), ((0, EDP - ED), (0, 0))).astype(bf16)
    wkT = jnp.pad(_head_pad_cols(wk.T), ((0, EDP - ED), (0, 0))).astype(bf16)
    wvT = jnp.pad(_head_pad_cols(wv.T), ((0, EDP - ED), (0, 0))).astype(bf16)
    bq = _head_pad_vec(qkv_b[0:ED])
    bk = _head_pad_vec(qkv_b[ED:2 * ED])
    bv = _head_pad_vec(qkv_b[2 * ED:3 * ED])
    alpha11 = alpha.reshape(1, 1)
    nw = norm_w.reshape(1, DIM)
    nb = norm_b.reshape(1, DIM)

    cw = _head_pad_cols(conv_w[:, 0, :].T)
    cw2 = jnp.concatenate([cw, cw], axis=1)
    sigb = jnp.broadcast_to(jax.nn.sigmoid(Bp).reshape(DS, 1), (DS, ED))
    sigb = jnp.concatenate([_head_pad_cols(sigb)] * B_SZ, axis=1)
    sigc = jnp.concatenate([_head_pad_cols(jax.nn.sigmoid(Cp).T)] * B_SZ,
                           axis=1)
    woT = _head_pad_cols(W_out).T.astype(bf16)
    bout = b_out.reshape(1, DIM)
    A = A.astype(bf16)

    out, ch2 = _run_front(x2d, winT, bin_p, wqT, wkT, wvT, bq, bk, bv,
                          alpha11, nw, nb)

    _, topk_idx = lax.top_k(ch2, K_CH)
    boff = jnp.arange(B_SZ, dtype=jnp.int32)[:, None] * L
    tpad = jnp.pad(topk_idx.astype(jnp.int32), ((0, 0), (0, K_PAD - K_CH)))
    tgt = tpad + boff
    valid = (jnp.arange(K_PAD, dtype=jnp.int32) < K_CH)[None, :]
    trash = BL + (jnp.arange(K_PAD, dtype=jnp.int32) % 128)[None, :]
    tgt = jnp.where(valid, tgt, trash)
    idx_g = (tpad + boff).reshape(NW * ROWS_W)
    halves = jnp.arange(2, dtype=jnp.int32)[:, None] * L
    tflat = tgt.reshape(1, -1)
    trash_flat = BL + (jnp.arange(NW * ROWS_W, dtype=jnp.int32) % 128)[None, :]
    own = (tflat >= halves) & (tflat < halves + L)
    idx_sc = jnp.where(own, tflat, trash_flat)
    idx_sc = idx_sc.reshape(B_SZ, 16, (NW * ROWS_W) // 16)

    sf, xres = _sc_gather(out, x2d, idx_g)

    xsum = _run_scan(sf, xres, cw2, sigb, sigc, A, woT, bout)

    res = _sc_scatter(xsum, idx_sc, x2d)
    return res[:BL].reshape(B_SZ, L, DIM)
